# baseline (speedup 1.0000x reference)
import functools

import jax
import jax.numpy as jnp
from jax import lax
from jax.experimental import pallas as pl
from jax.experimental.pallas import tpu as pltpu
from jax.experimental.pallas import tpu_sc as plsc

NSUB = 16
CH = 128
NB = 4
SB = 8
NBD = 8


def _ceil_div(a, b):
    return -(-a // b)


@functools.lru_cache(maxsize=None)
def _sc_scatter(w, k_chunks, np_rows):
    rpt = np_rows // NSUB
    sblocks = k_chunks // SB
    mesh = plsc.VectorSubcoreMesh(core_axis_name="c", subcore_axis_name="s")

    @functools.partial(
        pl.kernel,
        out_type=jax.ShapeDtypeStruct((2, np_rows, w), jnp.float32),
        mesh=mesh,
        scratch_types=[
            pltpu.VMEM((SB, CH), jnp.int32),
            pltpu.VMEM((SB, CH), jnp.int32),
            pltpu.VMEM((NB, CH, w), jnp.float32),
            pltpu.VMEM_SHARED((np_rows, w), jnp.float32),
            pltpu.SemaphoreType.DMA((SB,)),
            pltpu.SemaphoreType.DMA((SB,)),
            pltpu.SemaphoreType.DMA((NB,)),
            pltpu.SemaphoreType.DMA((NB,)),
        ],
        compiler_params=pltpu.CompilerParams(use_tc_tiling_on_sc=False),
    )
    def kern(src_hbm, dst_hbm, u_hbm, zeros_hbm, out_hbm,
             srcr, dstr, rows, acc, isem, jsem, gsem, ssem):
        cid = lax.axis_index("c")
        sid = lax.axis_index("s")
        pltpu.sync_copy(zeros_hbm, acc.at[pl.ds(sid * rpt, rpt)])
        plsc.subcore_barrier()

        def idx_load(c, slot):
            pltpu.async_copy(src_hbm.at[cid, sid, c], srcr.at[slot],
                             isem.at[slot])
            pltpu.async_copy(dst_hbm.at[cid, sid, c], dstr.at[slot],
                             jsem.at[slot])

        def idx_wait(c, slot):
            pltpu.make_async_copy(src_hbm.at[cid, sid, c], srcr.at[slot],
                                  isem.at[slot]).wait()
            pltpu.make_async_copy(dst_hbm.at[cid, sid, c], dstr.at[slot],
                                  jsem.at[slot]).wait()

        def gather_start(islot, rslot):
            pltpu.async_copy(u_hbm.at[srcr.at[islot]], rows.at[rslot],
                             gsem.at[rslot])

        def gather_wait(islot, rslot):
            pltpu.make_async_copy(u_hbm.at[srcr.at[islot]],
                                  rows.at[rslot], gsem.at[rslot]).wait()

        def scatter_start(islot, rslot):
            pltpu.async_copy(rows.at[rslot], acc.at[dstr.at[islot]],
                             ssem.at[rslot], add=True)

        def scatter_wait(islot, rslot):
            pltpu.make_async_copy(rows.at[rslot], acc.at[dstr.at[islot]],
                                  ssem.at[rslot]).wait()

        for j in range(SB):
            idx_load(j, j)
        for j in range(NB):
            idx_wait(j, j)
            gather_start(j, j)

        def sblock(sg, carry):
            base = sg * SB
            for j in range(NB):
                gather_wait(j, j)
                scatter_start(j, j)
            for j in range(NB):
                scatter_wait(j, j)
                idx_wait(base + NB + j, NB + j)
                gather_start(NB + j, j)

                @pl.when(base + SB + j < k_chunks)
                def _():
                    idx_load(base + SB + j, j)
            for j in range(NB):
                gather_wait(NB + j, j)
                scatter_start(NB + j, j)
            for j in range(NB):
                scatter_wait(NB + j, j)

                @pl.when(base + SB + j < k_chunks)
                def _():
                    idx_wait(base + SB + j, j)
                    gather_start(j, j)
                    pl.when(base + SB + NB + j < k_chunks)(
                        lambda: idx_load(base + SB + NB + j, NB + j))
            return carry

        lax.fori_loop(0, sblocks, sblock, 0)
        plsc.subcore_barrier()
        pltpu.sync_copy(acc.at[pl.ds(sid * rpt, rpt)],
                        out_hbm.at[cid, pl.ds(sid * rpt, rpt)])

    return kern


@functools.lru_cache(maxsize=None)
def _sc_degree(k_chunks, np_rows):
    rpt = np_rows // NSUB
    groups = k_chunks // NBD
    mesh = plsc.VectorSubcoreMesh(core_axis_name="c", subcore_axis_name="s")

    @functools.partial(
        pl.kernel,
        out_type=jax.ShapeDtypeStruct((2, np_rows, 8), jnp.float32),
        mesh=mesh,
        scratch_types=[
            pltpu.VMEM((k_chunks, CH), jnp.int32),
            pltpu.VMEM((CH, 8), jnp.float32),
            pltpu.VMEM_SHARED((np_rows, 8), jnp.float32),
            pltpu.SemaphoreType.DMA((NBD,)),
        ],
        compiler_params=pltpu.CompilerParams(use_tc_tiling_on_sc=False),
    )
    def kern(idx_hbm, zeros_hbm, ones_hbm, out_hbm, idxall, ones, acc, ssem):
        cid = lax.axis_index("c")
        sid = lax.axis_index("s")
        pltpu.sync_copy(idx_hbm.at[cid, sid], idxall)
        pltpu.sync_copy(ones_hbm, ones)
        pltpu.sync_copy(zeros_hbm, acc.at[pl.ds(sid * rpt, rpt)])
        plsc.subcore_barrier()

        def group(g, carry):
            base = g * NBD
            for b in range(NBD):
                @pl.when(g > 0)
                def _():
                    pltpu.make_async_copy(ones, acc.at[idxall.at[base + b]],
                                          ssem.at[b]).wait()
                pltpu.async_copy(ones, acc.at[idxall.at[base + b]],
                                 ssem.at[b], add=True)
            return carry

        lax.fori_loop(0, groups, group, 0)
        for b in range(NBD):
            pltpu.make_async_copy(ones, acc.at[idxall.at[b]],
                                  ssem.at[b]).wait()
        plsc.subcore_barrier()
        pltpu.sync_copy(acc.at[pl.ds(sid * rpt, rpt)],
                        out_hbm.at[cid, pl.ds(sid * rpt, rpt)])

    return kern


def _pack(col1, col2, k_chunks, fill, offset=0):
    total = NSUB * k_chunks * CH
    pad = total - col1.shape[0]
    fills = jnp.full((pad,), fill, col1.dtype)
    return jnp.stack([
        jnp.concatenate([col1, fills]).reshape(NSUB, k_chunks, CH),
        (jnp.concatenate([col2, fills]) + offset).reshape(
            NSUB, k_chunks, CH),
    ])


def _row(r, w):
    return pl.BlockSpec((r, w), lambda i: (i, 0))


def _full(shape):
    return pl.BlockSpec(shape, lambda i: tuple(0 for _ in shape))


def _relu(v):
    return jnp.maximum(v, 0.0)


def _dot(a, b):
    return jnp.dot(a, b, preferred_element_type=jnp.float32)


def _tc_prep(x, f, degp, Wx, bx, Wf1, bf1, Wf2, bf2, g, r, np_rows):
    def body(x_ref, f_ref, dg_ref, wx_ref, bx_ref, wf1_ref, bf1_ref,
             wf2_ref, bf2_ref, u0a_ref, u0b_ref, dinv_ref):
        dinv = lax.rsqrt(1.0 + dg_ref[0, :, 0:1])
        x0 = _relu(_dot(x_ref[...], wx_ref[...]) + bx_ref[...])
        f1 = _relu(_dot(f_ref[...], wf1_ref[...]) + bf1_ref[...])
        f0 = _relu(_dot(f1, wf2_ref[...]) + bf2_ref[...])
        u0a_ref[...] = dinv * x0
        u0b_ref[...] = dinv * f0
        dinv_ref[...] = dinv

    return pl.pallas_call(
        body,
        grid=(NSUB,),
        in_specs=[
            _row(r, 3), _row(r, 128),
            pl.BlockSpec((1, r, 8), lambda i: (g, i, 0)),
            _full((3, 32)), _full((1, 32)),
            _full((128, 64)), _full((1, 64)),
            _full((64, 16)), _full((1, 16)),
        ],
        out_specs=[_row(r, 32), _row(r, 16), _row(r, 1)],
        out_shape=[
            jax.ShapeDtypeStruct((np_rows, 32), jnp.float32),
            jax.ShapeDtypeStruct((np_rows, 16), jnp.float32),
            jax.ShapeDtypeStruct((np_rows, 1), jnp.float32),
        ],
    )(x, f, degp, Wx, bx.reshape(1, -1), Wf1, bf1.reshape(1, -1),
      Wf2, bf2.reshape(1, -1))


def _tc_layer(dinv, u_list, s_list, a_list, W, b, g, wout, r, np_rows,
              n_last=None):
    last = n_last is not None
    na = len(a_list)
    ns = len(s_list)
    s_widths = [s.shape[2] for s in s_list]
    a_widths = [a.shape[1] for a in a_list]
    wk = W.shape[0]
    akw = sum(s_widths)

    def body(*refs):
        dv_ref = refs[0]
        u_refs = refs[1:1 + ns]
        s_refs = refs[1 + ns:1 + 2 * ns]
        a_refs = refs[1 + 2 * ns:1 + 2 * ns + na]
        w_ref, b_ref = refs[1 + 2 * ns + na], refs[2 + 2 * ns + na]
        orefs = refs[3 + 2 * ns + na:]
        d = dv_ref[...]
        aks = [d * (s_ref[0] + u_ref[...])
               for s_ref, u_ref in zip(s_refs, u_refs)]
        acc = b_ref[...]
        off = 0
        for a_ref, wd in zip(a_refs, a_widths):
            acc = acc + _dot(a_ref[...], w_ref[off:off + wd, :])
            off += wd
        for ak in aks:
            acc = acc + _dot(ak, w_ref[off:off + ak.shape[1], :])
            off += ak.shape[1]
        hn = _relu(acc)
        if last:
            orefs[0][...] = hn
        else:
            ak_ref, un_ref = orefs
            aoff = 0
            for ak in aks:
                ak_ref[:, aoff:aoff + ak.shape[1]] = ak
                aoff += ak.shape[1]
            un_ref[...] = d * hn

    in_specs = ([_row(r, 1)]
                + [_row(r, wd) for wd in s_widths]
                + [pl.BlockSpec((1, r, wd), lambda i, g=g: (g, i, 0))
                   for wd in s_widths]
                + [_row(r, wd) for wd in a_widths]
                + [_full((wk, wout)), _full((1, wout))])
    if last:
        out_specs = _row(r, wout)
        out_shape = jax.ShapeDtypeStruct((n_last, wout), jnp.float32)
    else:
        out_specs = [_row(r, akw), _row(r, wout)]
        out_shape = [
            jax.ShapeDtypeStruct((np_rows, akw), jnp.float32),
            jax.ShapeDtypeStruct((np_rows, wout), jnp.float32),
        ]
    return pl.pallas_call(
        body,
        grid=(NSUB,),
        in_specs=in_specs,
        out_specs=out_specs,
        out_shape=out_shape,
    )(dinv, *u_list, *s_list, *a_list, W, b.reshape(1, -1))


def kernel(x1, f1, edge_index1, g1, x2, f2, edge_index2, g2,
           Wx, bx, Wf1, bf1, Wf2, bf2, W1, b1, W2, b2, W3, b3, W4, b4):
    n = x1.shape[0]
    e = edge_index1.shape[1]
    np_rows = _ceil_div(n, NSUB * CH) * NSUB * CH
    r = np_rows // NSUB
    k_ch = _ceil_div(e, NSUB * CH * SB) * SB
    k_deg = _ceil_div(e, NSUB * CH * NBD) * NBD

    src_p = _pack(edge_index1[0], edge_index2[0], k_ch, 0, offset=np_rows)
    dst_p = _pack(edge_index1[1], edge_index2[1], k_ch, np_rows - 1)
    deg_p = _pack(edge_index1[1], edge_index2[1], k_deg, np_rows - 1)

    zeros8 = jnp.zeros((r, 8), jnp.float32)
    ones8 = jnp.ones((CH, 8), jnp.float32)
    zeros16 = jnp.zeros((r, 16), jnp.float32)
    zeros32 = jnp.zeros((r, 32), jnp.float32)

    degp = _sc_degree(k_deg, np_rows)(deg_p, zeros8, ones8)

    scat32 = _sc_scatter(32, k_ch, np_rows)
    scat16 = _sc_scatter(16, k_ch, np_rows)

    ua, ub, dv = [], [], []
    for g, (x, f) in enumerate([(x1, f1), (x2, f2)]):
        u0a, u0b, dinv = _tc_prep(x, f, degp, Wx, bx, Wf1, bf1, Wf2, bf2,
                                  g, r, np_rows)
        ua.append(u0a), ub.append(u0b), dv.append(dinv)

    s0a = scat32(src_p, dst_p, jnp.concatenate(ua), zeros32)
    s0b = scat16(src_p, dst_p, jnp.concatenate(ub), zeros16)

    a0s, u1s = [], []
    for g in range(2):
        a0, u1 = _tc_layer(dv[g], [ua[g], ub[g]], [s0a, s0b], [],
                           W1, b1, g, 32, r, np_rows)
        a0s.append(a0), u1s.append(u1)

    s1 = scat32(src_p, dst_p, jnp.concatenate(u1s), zeros32)
    a1s, u2s = [], []
    for g in range(2):
        a1, u2 = _tc_layer(dv[g], [u1s[g]], [s1], [a0s[g]],
                           W2, b2, g, 32, r, np_rows)
        a1s.append(a1), u2s.append(u2)

    s2 = scat32(src_p, dst_p, jnp.concatenate(u2s), zeros32)
    a2s, u3s = [], []
    for g in range(2):
        a2, u3 = _tc_layer(dv[g], [u2s[g]], [s2], [a0s[g], a1s[g]],
                           W3, b3, g, 32, r, np_rows)
        a2s.append(a2), u3s.append(u3)

    s3 = scat32(src_p, dst_p, jnp.concatenate(u3s), zeros32)
    outs = []
    for g in range(2):
        h4 = _tc_layer(dv[g], [u3s[g]], [s3], [a0s[g], a1s[g], a2s[g]],
                       W4, b4, g, 64, r, np_rows, n_last=n)
        outs.append(h4)

    return (outs[0], outs[1], g1, g2)


# speedup vs baseline: 1.0687x; 1.0687x over previous
import functools

import jax
import jax.numpy as jnp
from jax import lax
from jax.experimental import pallas as pl
from jax.experimental.pallas import tpu as pltpu
from jax.experimental.pallas import tpu_sc as plsc

NSUB = 16
CH = 128
NB = 4
SB = 8
BPGP = 16
BPGL = 8


def _ceil_div(a, b):
    return -(-a // b)


@functools.lru_cache(maxsize=None)
def _sc_scatter(w, k_chunks, np_rows):
    rpt = np_rows // NSUB
    sblocks = k_chunks // SB
    mesh = plsc.VectorSubcoreMesh(core_axis_name="c", subcore_axis_name="s")

    @functools.partial(
        pl.kernel,
        out_type=jax.ShapeDtypeStruct((2, np_rows, w), jnp.float32),
        mesh=mesh,
        scratch_types=[
            pltpu.VMEM((SB, CH), jnp.int32),
            pltpu.VMEM((SB, CH), jnp.int32),
            pltpu.VMEM((NB, CH, w), jnp.float32),
            pltpu.VMEM_SHARED((np_rows, w), jnp.float32),
            pltpu.SemaphoreType.DMA((SB,)),
            pltpu.SemaphoreType.DMA((SB,)),
            pltpu.SemaphoreType.DMA((NB,)),
            pltpu.SemaphoreType.DMA((NB,)),
        ],
        compiler_params=pltpu.CompilerParams(use_tc_tiling_on_sc=False),
    )
    def kern(idx_hbm, u_hbm, zeros_hbm, out_hbm,
             srcr, dstr, rows, acc, isem, jsem, gsem, ssem):
        cid = lax.axis_index("c")
        sid = lax.axis_index("s")
        pltpu.sync_copy(zeros_hbm, acc.at[pl.ds(sid * rpt, rpt)])
        plsc.subcore_barrier()

        def idx_load(c, slot):
            pltpu.async_copy(idx_hbm.at[0, cid, sid, c], srcr.at[slot],
                             isem.at[slot])
            pltpu.async_copy(idx_hbm.at[1, cid, sid, c], dstr.at[slot],
                             jsem.at[slot])

        def idx_wait(c, slot):
            pltpu.make_async_copy(idx_hbm.at[0, cid, sid, c], srcr.at[slot],
                                  isem.at[slot]).wait()
            pltpu.make_async_copy(idx_hbm.at[1, cid, sid, c], dstr.at[slot],
                                  jsem.at[slot]).wait()

        def gather_start(islot, rslot):
            pltpu.async_copy(u_hbm.at[srcr.at[islot]], rows.at[rslot],
                             gsem.at[rslot])

        def gather_wait(islot, rslot):
            pltpu.make_async_copy(u_hbm.at[srcr.at[islot]],
                                  rows.at[rslot], gsem.at[rslot]).wait()

        def scatter_start(islot, rslot):
            pltpu.async_copy(rows.at[rslot], acc.at[dstr.at[islot]],
                             ssem.at[rslot], add=True)

        def scatter_wait(islot, rslot):
            pltpu.make_async_copy(rows.at[rslot], acc.at[dstr.at[islot]],
                                  ssem.at[rslot]).wait()

        for j in range(SB):
            idx_load(j, j)
        for j in range(NB):
            idx_wait(j, j)
            gather_start(j, j)

        def sblock(sg, carry):
            base = sg * SB
            for j in range(NB):
                gather_wait(j, j)
                scatter_start(j, j)
            for j in range(NB):
                scatter_wait(j, j)
                idx_wait(base + NB + j, NB + j)
                gather_start(NB + j, j)

                @pl.when(base + SB + j < k_chunks)
                def _():
                    idx_load(base + SB + j, j)
            for j in range(NB):
                gather_wait(NB + j, j)
                scatter_start(NB + j, j)
            for j in range(NB):
                scatter_wait(NB + j, j)

                @pl.when(base + SB + j < k_chunks)
                def _():
                    idx_wait(base + SB + j, j)
                    gather_start(j, j)
                    pl.when(base + SB + NB + j < k_chunks)(
                        lambda: idx_load(base + SB + NB + j, NB + j))
            return carry

        lax.fori_loop(0, sblocks, sblock, 0)
        plsc.subcore_barrier()
        pltpu.sync_copy(acc.at[pl.ds(sid * rpt, rpt)],
                        out_hbm.at[cid, pl.ds(sid * rpt, rpt)])

    return kern


@functools.lru_cache(maxsize=None)
def _sc_degree(k_chunks, np_rows):
    rpt = np_rows // NSUB
    groups = k_chunks // SB
    mesh = plsc.VectorSubcoreMesh(core_axis_name="c", subcore_axis_name="s")

    @functools.partial(
        pl.kernel,
        out_type=jax.ShapeDtypeStruct((2, np_rows, 8), jnp.float32),
        mesh=mesh,
        scratch_types=[
            pltpu.VMEM((k_chunks, CH), jnp.int32),
            pltpu.VMEM((CH, 8), jnp.float32),
            pltpu.VMEM_SHARED((np_rows, 8), jnp.float32),
            pltpu.SemaphoreType.DMA((SB,)),
        ],
        compiler_params=pltpu.CompilerParams(use_tc_tiling_on_sc=False),
    )
    def kern(idx_hbm, zeros_hbm, ones_hbm, out_hbm, idxall, ones, acc, ssem):
        cid = lax.axis_index("c")
        sid = lax.axis_index("s")
        pltpu.sync_copy(idx_hbm.at[1, cid, sid], idxall)
        pltpu.sync_copy(ones_hbm, ones)
        pltpu.sync_copy(zeros_hbm, acc.at[pl.ds(sid * rpt, rpt)])
        plsc.subcore_barrier()

        def group(g, carry):
            base = g * SB
            for b in range(SB):
                @pl.when(g > 0)
                def _():
                    pltpu.make_async_copy(ones, acc.at[idxall.at[base + b]],
                                          ssem.at[b]).wait()
                pltpu.async_copy(ones, acc.at[idxall.at[base + b]],
                                 ssem.at[b], add=True)
            return carry

        lax.fori_loop(0, groups, group, 0)
        for b in range(SB):
            pltpu.make_async_copy(ones, acc.at[idxall.at[b]],
                                  ssem.at[b]).wait()
        plsc.subcore_barrier()
        pltpu.sync_copy(acc.at[pl.ds(sid * rpt, rpt)],
                        out_hbm.at[cid, pl.ds(sid * rpt, rpt)])

    return kern


def _full(shape):
    return pl.BlockSpec(shape, lambda g, i: tuple(0 for _ in shape))


def _relu(v):
    return jnp.maximum(v, 0.0)


def _dot(a, b):
    return jnp.dot(a, b, preferred_element_type=jnp.float32)


def _tc_prep(x, f, degp, Wx, bx, Wf1, bf1, Wf2, bf2, np_rows):
    r = np_rows // BPGP
    def body(x_ref, f_ref, dg_ref, wx_ref, bx_ref, wf1_ref, bf1_ref,
             wf2_ref, bf2_ref, u0a_ref, u0b_ref, dinv_ref):
        dinv = lax.rsqrt(1.0 + dg_ref[0, :, 0:1])
        x0 = _relu(_dot(x_ref[0], wx_ref[...]) + bx_ref[...])
        f1 = _relu(_dot(f_ref[0], wf1_ref[...]) + bf1_ref[...])
        f0 = _relu(_dot(f1, wf2_ref[...]) + bf2_ref[...])
        u0a_ref[...] = dinv * x0
        u0b_ref[...] = dinv * f0
        dinv_ref[0] = dinv

    def spec3(w):
        return pl.BlockSpec((1, r, w), lambda g, i: (g, i, 0))

    def tab(w):
        return pl.BlockSpec((r, w), lambda g, i: (g * BPGP + i, 0))

    return pl.pallas_call(
        body,
        grid=(2, BPGP),
        in_specs=[
            spec3(3), spec3(128), spec3(8),
            _full((3, 32)), _full((1, 32)),
            _full((128, 64)), _full((1, 64)),
            _full((64, 16)), _full((1, 16)),
        ],
        out_specs=[tab(32), tab(16), spec3(1)],
        out_shape=[
            jax.ShapeDtypeStruct((2 * np_rows, 32), jnp.float32),
            jax.ShapeDtypeStruct((2 * np_rows, 16), jnp.float32),
            jax.ShapeDtypeStruct((2, np_rows, 1), jnp.float32),
        ],
    )(x, f, degp, Wx, bx.reshape(1, -1), Wf1, bf1.reshape(1, -1),
      Wf2, bf2.reshape(1, -1))


def _tc_layer(dinv, u_list, s_list, a_list, W, b, wout, np_rows):
    r = np_rows // BPGL
    na = len(a_list)
    ns = len(s_list)
    s_widths = [s.shape[2] for s in s_list]
    a_widths = [a.shape[2] for a in a_list]
    wk = W.shape[0]
    akw = sum(s_widths)

    def body(*refs):
        dv_ref = refs[0]
        u_refs = refs[1:1 + ns]
        s_refs = refs[1 + ns:1 + 2 * ns]
        a_refs = refs[1 + 2 * ns:1 + 2 * ns + na]
        w_ref, b_ref = refs[1 + 2 * ns + na], refs[2 + 2 * ns + na]
        ak_ref, un_ref = refs[3 + 2 * ns + na:]
        d = dv_ref[0]
        aks = [d * (s_ref[0] + u_ref[...])
               for s_ref, u_ref in zip(s_refs, u_refs)]
        acc = b_ref[...]
        off = 0
        for a_ref, wd in zip(a_refs, a_widths):
            acc = acc + _dot(a_ref[0], w_ref[off:off + wd, :])
            off += wd
        for ak in aks:
            acc = acc + _dot(ak, w_ref[off:off + ak.shape[1], :])
            off += ak.shape[1]
        aoff = 0
        for ak in aks:
            ak_ref[0, :, aoff:aoff + ak.shape[1]] = ak
            aoff += ak.shape[1]
        un_ref[...] = d * _relu(acc)

    def spec3(w):
        return pl.BlockSpec((1, r, w), lambda g, i: (g, i, 0))

    def tab(w):
        return pl.BlockSpec((r, w), lambda g, i: (g * BPGL + i, 0))

    in_specs = ([spec3(1)]
                + [tab(wd) for wd in s_widths]
                + [spec3(wd) for wd in s_widths]
                + [spec3(wd) for wd in a_widths]
                + [_full((wk, wout)), _full((1, wout))])
    return pl.pallas_call(
        body,
        grid=(2, BPGL),
        in_specs=in_specs,
        out_specs=[spec3(akw), tab(wout)],
        out_shape=[
            jax.ShapeDtypeStruct((2, np_rows, akw), jnp.float32),
            jax.ShapeDtypeStruct((2 * np_rows, wout), jnp.float32),
        ],
    )(dinv, *u_list, *s_list, *a_list, W, b.reshape(1, -1))


def _tc_final(dinv, u3, s3, a_list, W, b, g, n, np_rows):
    r = np_rows // BPGL
    def body(dv_ref, u_ref, s_ref, a0_ref, a1_ref, a2_ref, w_ref, b_ref,
             out_ref):
        d = dv_ref[0]
        a3 = d * (s_ref[0] + u_ref[...])
        acc = (b_ref[...]
               + _dot(a0_ref[0], w_ref[0:48, :])
               + _dot(a1_ref[0], w_ref[48:80, :])
               + _dot(a2_ref[0], w_ref[80:112, :])
               + _dot(a3, w_ref[112:144, :]))
        out_ref[...] = _relu(acc)

    def spec3(w):
        return pl.BlockSpec((1, r, w), lambda i, g=g: (g, i, 0))

    return pl.pallas_call(
        body,
        grid=(BPGL,),
        in_specs=[
            spec3(1),
            pl.BlockSpec((r, 32), lambda i, g=g: (g * BPGL + i, 0)),
            spec3(32), spec3(48), spec3(32), spec3(32),
            pl.BlockSpec((144, 64), lambda i: (0, 0)),
            pl.BlockSpec((1, 64), lambda i: (0, 0)),
        ],
        out_specs=pl.BlockSpec((r, 64), lambda i: (i, 0)),
        out_shape=jax.ShapeDtypeStruct((n, 64), jnp.float32),
    )(dinv, u3, s3, *a_list, W, b.reshape(1, -1))


def kernel(x1, f1, edge_index1, g1, x2, f2, edge_index2, g2,
           Wx, bx, Wf1, bf1, Wf2, bf2, W1, b1, W2, b2, W3, b3, W4, b4):
    n = x1.shape[0]
    e = edge_index1.shape[1]
    np_rows = _ceil_div(n, NSUB * CH) * NSUB * CH
    rpt = np_rows // NSUB
    k_ch = _ceil_div(e, NSUB * CH * SB) * SB

    total = NSUB * k_ch * CH
    pad = total - e
    f0 = jnp.zeros((pad,), jnp.int32)
    fnp = jnp.full((pad,), np_rows, jnp.int32)
    ftr = jnp.full((pad,), np_rows - 1, jnp.int32)
    idx_p = jnp.concatenate([
        edge_index1[0], f0, edge_index2[0] + np_rows, fnp,
        edge_index1[1], ftr, edge_index2[1], ftr,
    ]).reshape(2, 2, NSUB, k_ch, CH)

    zeros8 = jnp.zeros((rpt, 8), jnp.float32)
    ones8 = jnp.ones((CH, 8), jnp.float32)
    zeros16 = jnp.zeros((rpt, 16), jnp.float32)
    zeros32 = jnp.zeros((rpt, 32), jnp.float32)

    x = jnp.stack([x1, x2])
    f = jnp.stack([f1, f2])

    degp = _sc_degree(k_ch, np_rows)(idx_p, zeros8, ones8)
    scat32 = _sc_scatter(32, k_ch, np_rows)
    scat16 = _sc_scatter(16, k_ch, np_rows)

    u0a, u0b, dinv = _tc_prep(x, f, degp, Wx, bx, Wf1, bf1, Wf2, bf2,
                              np_rows)
    s0a = scat32(idx_p, u0a, zeros32)
    s0b = scat16(idx_p, u0b, zeros16)
    a0, u1 = _tc_layer(dinv, [u0a, u0b], [s0a, s0b], [], W1, b1, 32,
                       np_rows)
    s1 = scat32(idx_p, u1, zeros32)
    a1, u2 = _tc_layer(dinv, [u1], [s1], [a0], W2, b2, 32, np_rows)
    s2 = scat32(idx_p, u2, zeros32)
    a2, u3 = _tc_layer(dinv, [u2], [s2], [a0, a1], W3, b3, 32, np_rows)
    s3 = scat32(idx_p, u3, zeros32)
    outs = [_tc_final(dinv, u3, s3, [a0, a1, a2], W4, b4, g, n, np_rows)
            for g in range(2)]

    return (outs[0], outs[1], g1, g2)


# speedup vs baseline: 1.3537x; 1.2667x over previous
import functools

import jax
import jax.numpy as jnp
from jax import lax
from jax.experimental import pallas as pl
from jax.experimental.pallas import tpu as pltpu
from jax.experimental.pallas import tpu_sc as plsc

NSUB = 16
CH = 128
NB = 4
SB = 8
BP = 4


def _ceil_div(a, b):
    return -(-a // b)


@functools.lru_cache(maxsize=None)
def _sc_scatter(k_chunks, np_rows):
    w = 32
    rpt = np_rows // NSUB
    cpt = k_chunks * CH
    sblocks = k_chunks // SB
    mesh = plsc.VectorSubcoreMesh(core_axis_name="c", subcore_axis_name="s")

    @functools.partial(
        pl.kernel,
        out_type=jax.ShapeDtypeStruct((2, np_rows, w), jnp.float32),
        mesh=mesh,
        scratch_types=[
            pltpu.VMEM((SB, CH), jnp.int32),
            pltpu.VMEM((SB, CH), jnp.int32),
            pltpu.VMEM((NB, CH, w), jnp.float32),
            pltpu.VMEM_SHARED((np_rows, w), jnp.float32),
            pltpu.SemaphoreType.DMA((SB,)),
            pltpu.SemaphoreType.DMA((SB,)),
            pltpu.SemaphoreType.DMA((NB,)),
            pltpu.SemaphoreType.DMA((NB,)),
        ],
        compiler_params=pltpu.CompilerParams(use_tc_tiling_on_sc=False),
    )
    def kern(eidx_hbm, u_hbm, zeros_hbm, out_hbm,
             srcr, dstr, rows, acc, isem, jsem, gsem, ssem):
        cid = lax.axis_index("c")
        sid = lax.axis_index("s")
        uoff = cid * np_rows
        pltpu.sync_copy(zeros_hbm, acc.at[pl.ds(sid * rpt, rpt)])
        plsc.subcore_barrier()

        def idx_load(c, slot):
            off = sid * cpt + c * CH
            pltpu.async_copy(eidx_hbm.at[cid, 0, pl.ds(off, CH)],
                             srcr.at[slot], isem.at[slot])
            pltpu.async_copy(eidx_hbm.at[cid, 1, pl.ds(off, CH)],
                             dstr.at[slot], jsem.at[slot])

        def idx_wait(c, slot):
            off = sid * cpt + c * CH
            pltpu.make_async_copy(eidx_hbm.at[cid, 0, pl.ds(off, CH)],
                                  srcr.at[slot], isem.at[slot]).wait()
            pltpu.make_async_copy(eidx_hbm.at[cid, 1, pl.ds(off, CH)],
                                  dstr.at[slot], jsem.at[slot]).wait()
            for kk in range(CH // 16):
                sl = pl.ds(kk * 16, 16)
                srcr[slot, sl] = srcr[slot, sl] + uoff

        def gather_start(islot, rslot):
            pltpu.async_copy(u_hbm.at[srcr.at[islot]], rows.at[rslot],
                             gsem.at[rslot])

        def gather_wait(islot, rslot):
            pltpu.make_async_copy(u_hbm.at[srcr.at[islot]],
                                  rows.at[rslot], gsem.at[rslot]).wait()

        def scatter_start(islot, rslot):
            pltpu.async_copy(rows.at[rslot], acc.at[dstr.at[islot]],
                             ssem.at[rslot], add=True)

        def scatter_wait(islot, rslot):
            pltpu.make_async_copy(rows.at[rslot], acc.at[dstr.at[islot]],
                                  ssem.at[rslot]).wait()

        for j in range(SB):
            idx_load(j, j)
        for j in range(NB):
            idx_wait(j, j)
            gather_start(j, j)

        def sblock(sg, carry):
            base = sg * SB
            for j in range(NB):
                gather_wait(j, j)
                scatter_start(j, j)
            for j in range(NB):
                scatter_wait(j, j)
                idx_wait(base + NB + j, NB + j)
                gather_start(NB + j, j)

                @pl.when(base + SB + j < k_chunks)
                def _():
                    idx_load(base + SB + j, j)
            for j in range(NB):
                gather_wait(NB + j, j)
                scatter_start(NB + j, j)
            for j in range(NB):
                scatter_wait(NB + j, j)

                @pl.when(base + SB + j < k_chunks)
                def _():
                    idx_wait(base + SB + j, j)
                    gather_start(j, j)
                    pl.when(base + SB + NB + j < k_chunks)(
                        lambda: idx_load(base + SB + NB + j, NB + j))
            return carry

        lax.fori_loop(0, sblocks, sblock, 0)
        plsc.subcore_barrier()
        pltpu.sync_copy(acc.at[pl.ds(sid * rpt, rpt)],
                        out_hbm.at[cid, pl.ds(sid * rpt, rpt)])

    return kern


@functools.lru_cache(maxsize=None)
def _sc_degree(k_chunks, np_rows):
    w = 32
    rpt = np_rows // NSUB
    cpt = k_chunks * CH
    sblocks = k_chunks // SB
    mesh = plsc.VectorSubcoreMesh(core_axis_name="c", subcore_axis_name="s")

    @functools.partial(
        pl.kernel,
        out_type=jax.ShapeDtypeStruct((2, np_rows, w), jnp.float32),
        mesh=mesh,
        scratch_types=[
            pltpu.VMEM((SB, CH), jnp.int32),
            pltpu.VMEM((CH, w), jnp.float32),
            pltpu.VMEM_SHARED((np_rows, w), jnp.float32),
            pltpu.SemaphoreType.DMA((SB,)),
            pltpu.SemaphoreType.DMA((SB,)),
        ],
        compiler_params=pltpu.CompilerParams(use_tc_tiling_on_sc=False),
    )
    def kern(eidx_hbm, zeros_hbm, ones_hbm, out_hbm,
             dstr, ones, acc, isem, ssem):
        cid = lax.axis_index("c")
        sid = lax.axis_index("s")
        pltpu.sync_copy(ones_hbm, ones)
        pltpu.sync_copy(zeros_hbm, acc.at[pl.ds(sid * rpt, rpt)])
        plsc.subcore_barrier()

        def idx_load(c, slot):
            off = sid * cpt + c * CH
            pltpu.async_copy(eidx_hbm.at[cid, 1, pl.ds(off, CH)],
                             dstr.at[slot], isem.at[slot])

        def idx_wait(c, slot):
            off = sid * cpt + c * CH
            pltpu.make_async_copy(eidx_hbm.at[cid, 1, pl.ds(off, CH)],
                                  dstr.at[slot], isem.at[slot]).wait()

        def sblock(sg, carry):
            base = sg * SB
            for j in range(SB):
                @pl.when(sg > 0)
                def _():
                    pltpu.make_async_copy(ones, acc.at[dstr.at[j]],
                                          ssem.at[j]).wait()
                idx_load(base + j, j)
            for j in range(SB):
                idx_wait(base + j, j)
                pltpu.async_copy(ones, acc.at[dstr.at[j]], ssem.at[j],
                                 add=True)
            return carry

        lax.fori_loop(0, sblocks, sblock, 0)
        for j in range(SB):
            pltpu.make_async_copy(ones, acc.at[dstr.at[j]],
                                  ssem.at[j]).wait()
        plsc.subcore_barrier()
        pltpu.sync_copy(acc.at[pl.ds(sid * rpt, rpt)],
                        out_hbm.at[cid, pl.ds(sid * rpt, rpt)])

    return kern


def _full(shape):
    return pl.BlockSpec(shape, lambda *_: tuple(0 for _ in shape))


def _relu(v):
    return jnp.maximum(v, 0.0)


def _dot(a, b):
    return jnp.dot(a, b, preferred_element_type=jnp.float32)


def _pdot(ap, W, span=32):
    kin = W.shape[0]
    return jnp.concatenate(
        [_dot(ap[:, span * j:span * j + kin], W) for j in range(4)], axis=1)


def _tile4(b):
    return jnp.concatenate([b, b, b, b], axis=1)


def _tc_prep(xp, fp, degp, Wx, bx, Wf1, bf1, Wf2, bf2, n4, np4):
    r = np4 // BP

    def body(x_ref, f_ref, dg_ref, wx_ref, bx_ref, wf1_ref, bf1_ref,
             wf2_ref, bf2_ref, u0a_ref, u0b_ref, d32_ref):
        d32 = lax.rsqrt(1.0 + dg_ref[0])
        bx4 = _tile4(bx_ref[...])
        x0 = _relu(_pdot(x_ref[0], wx_ref[...], span=3) + bx4)
        bf14 = _tile4(bf1_ref[...])
        f1 = _relu(_pdot(f_ref[0], wf1_ref[...], span=128) + bf14)
        z16 = jnp.zeros((f1.shape[0], 16), jnp.float32)
        f0parts = []
        for j in range(4):
            f0parts.append(_relu(_dot(f1[:, 64 * j:64 * j + 64],
                                      wf2_ref[...]) + bf2_ref[...]))
            f0parts.append(z16)
        f0 = jnp.concatenate(f0parts, axis=1)
        u0a_ref[...] = d32 * x0
        u0b_ref[...] = d32 * f0
        d32_ref[0] = d32

    def spec3(w):
        return pl.BlockSpec((1, r, w), lambda g, i: (g, i, 0))

    def tab(w):
        return pl.BlockSpec((r, w), lambda g, i: (g * BP + i, 0))

    return pl.pallas_call(
        body,
        grid=(2, BP),
        in_specs=[
            spec3(12), spec3(512), spec3(128),
            _full((3, 32)), _full((1, 32)),
            _full((128, 64)), _full((1, 64)),
            _full((64, 16)), _full((1, 16)),
        ],
        out_specs=[tab(128), tab(128), spec3(128)],
        out_shape=[
            jax.ShapeDtypeStruct((2 * np4, 128), jnp.float32),
            jax.ShapeDtypeStruct((2 * np4, 128), jnp.float32),
            jax.ShapeDtypeStruct((2, np4, 128), jnp.float32),
        ],
    )(xp, fp, degp, Wx, bx.reshape(1, -1), Wf1, bf1.reshape(1, -1),
      Wf2, bf2.reshape(1, -1))


def _tc_layer(d32, us_pairs, a_list, W, wblocks, b, np4):
    r = np4 // BP
    npairs = len(us_pairs)
    na = len(a_list)

    def body(*refs):
        d_ref = refs[0]
        u_refs = refs[1:1 + npairs]
        s_refs = refs[1 + npairs:1 + 2 * npairs]
        a_refs = refs[1 + 2 * npairs:1 + 2 * npairs + na]
        w_ref, b_ref = refs[1 + 2 * npairs + na], refs[2 + 2 * npairs + na]
        ak_refs = refs[3 + 2 * npairs + na:3 + 2 * npairs + na + npairs]
        un_ref = refs[3 + 2 * npairs + na + npairs]
        d = d_ref[0]
        aks = [d * (s_ref[0] + u_ref[...])
               for s_ref, u_ref in zip(s_refs, u_refs)]
        acc = _tile4(b_ref[...])
        ops = [a_ref[0] for a_ref in a_refs] + aks
        for opv, (kin, off) in zip(ops, wblocks):
            acc = acc + _pdot(opv, w_ref[off:off + kin, :])
        for ak, ak_ref in zip(aks, ak_refs):
            ak_ref[0] = ak
        un_ref[...] = d * _relu(acc)

    def spec3(w):
        return pl.BlockSpec((1, r, w), lambda g, i: (g, i, 0))

    def tab(w):
        return pl.BlockSpec((r, w), lambda g, i: (g * BP + i, 0))

    wk = W.shape[0]
    wout = W.shape[1]
    in_specs = ([spec3(128)]
                + [tab(128)] * npairs
                + [spec3(128)] * npairs
                + [spec3(128)] * na
                + [_full((wk, wout)), _full((1, wout))])
    out_specs = [spec3(128)] * npairs + [tab(4 * wout)]
    out_shape = ([jax.ShapeDtypeStruct((2, np4, 128), jnp.float32)] * npairs
                 + [jax.ShapeDtypeStruct((2 * np4, 4 * wout), jnp.float32)])
    us = [p[0] for p in us_pairs]
    ss = [p[1] for p in us_pairs]
    res = pl.pallas_call(
        body,
        grid=(2, BP),
        in_specs=in_specs,
        out_specs=out_specs,
        out_shape=out_shape,
    )(d32, *us, *ss, *a_list, W, b.reshape(1, -1))
    return res[:npairs], res[npairs]


def _tc_final(d32, u3, s3, a_list, W, b, g, n4, np4):
    r = np4 // BP

    def body(d_ref, u_ref, s_ref, a0a_ref, a0b_ref, a1_ref, a2_ref,
             w_ref, b_ref, out_ref):
        d = d_ref[0]
        a3 = d * (s_ref[0] + u_ref[...])
        acc = _tile4(b_ref[...])
        acc = acc + _pdot(a0a_ref[0], w_ref[0:32, :])
        acc = acc + _pdot(a0b_ref[0], w_ref[32:48, :])
        acc = acc + _pdot(a1_ref[0], w_ref[48:80, :])
        acc = acc + _pdot(a2_ref[0], w_ref[80:112, :])
        acc = acc + _pdot(a3, w_ref[112:144, :])
        out_ref[...] = _relu(acc)

    def spec3(w):
        return pl.BlockSpec((1, r, w), lambda i, g=g: (g, i, 0))

    return pl.pallas_call(
        body,
        grid=(BP,),
        in_specs=[
            spec3(128),
            pl.BlockSpec((r, 128), lambda i, g=g: (g * BP + i, 0)),
            spec3(128), spec3(128), spec3(128), spec3(128), spec3(128),
            pl.BlockSpec((144, 64), lambda i: (0, 0)),
            pl.BlockSpec((1, 64), lambda i: (0, 0)),
        ],
        out_specs=pl.BlockSpec((r, 256), lambda i: (i, 0)),
        out_shape=jax.ShapeDtypeStruct((n4, 256), jnp.float32),
    )(d32, u3, s3, *a_list, W, b.reshape(1, -1))


def kernel(x1, f1, edge_index1, g1, x2, f2, edge_index2, g2,
           Wx, bx, Wf1, bf1, Wf2, bf2, W1, b1, W2, b2, W3, b3, W4, b4):
    n = x1.shape[0]
    e = edge_index1.shape[1]
    np_rows = _ceil_div(n, NSUB * CH) * NSUB * CH
    np4 = np_rows // 4
    n4 = n // 4
    rpt = np_rows // NSUB
    k_ch = _ceil_div(e, NSUB * CH * SB) * SB
    e_pad = NSUB * k_ch * CH

    padb = jnp.stack([jnp.zeros((e_pad - e,), jnp.int32),
                      jnp.full((e_pad - e,), np_rows - 1, jnp.int32)])
    eidx = jnp.stack([jnp.concatenate([edge_index1, padb], axis=1),
                      jnp.concatenate([edge_index2, padb], axis=1)])

    xp = jnp.stack([x1, x2]).reshape(2, n4, 12)
    fp = jnp.stack([f1, f2]).reshape(2, n4, 512)
    zeros32 = jnp.zeros((rpt, 32), jnp.float32)
    ones32 = jnp.ones((CH, 32), jnp.float32)

    def packed(s):
        return s.reshape(2, np4, 128)

    degp = packed(_sc_degree(k_ch, np_rows)(eidx, zeros32, ones32))
    scat = _sc_scatter(k_ch, np_rows)

    def flat(u):
        return u.reshape(2 * np_rows, 32)

    u0a, u0b, d32 = _tc_prep(xp, fp, degp, Wx, bx, Wf1, bf1, Wf2, bf2,
                             n4, np4)
    s0a = packed(scat(eidx, flat(u0a), zeros32))
    s0b = packed(scat(eidx, flat(u0b), zeros32))
    (a0a, a0b), u1 = _tc_layer(d32, [(u0a, s0a), (u0b, s0b)], [],
                               W1, [(32, 0), (16, 32)], b1, np4)
    s1 = packed(scat(eidx, flat(u1), zeros32))
    (a1,), u2 = _tc_layer(d32, [(u1, s1)], [a0a, a0b],
                          W2, [(32, 0), (16, 32), (32, 48)], b2, np4)
    s2 = packed(scat(eidx, flat(u2), zeros32))
    (a2,), u3 = _tc_layer(d32, [(u2, s2)], [a0a, a0b, a1],
                          W3, [(32, 0), (16, 32), (32, 48), (32, 80)],
                          b3, np4)
    s3 = packed(scat(eidx, flat(u3), zeros32))
    outs = [_tc_final(d32, u3, s3, [a0a, a0b, a1, a2], W4, b4, g, n4, np4)
            .reshape(n, 64) for g in range(2)]

    return (outs[0], outs[1], g1, g2)
